# baseline (speedup 1.0000x reference)
import functools

import jax
import jax.numpy as jnp
from jax import lax
from jax.experimental import pallas as pl
from jax.experimental.pallas import tpu as pltpu
from jax.experimental.pallas import tpu_sc as plsc

T = 32
B = 4096
D = 128
TB = T * B
MB_ROWS = TB // 4

NC = 2
NS = 16
NW = NC * NS
N_PER_W = MB_ROWS // NW
C = 128
NCHUNK = N_PER_W // C
NBUF = 7
LANES = 16
IGNORED = -1


def _body(mem_hbm, val_hbm, lo_hbm, idx_hbm, out_hbm,
          idxm_v, idxv_v, lo_v, *rest):
    bufs = rest[:NBUF]
    gm_sems = rest[NBUF:2 * NBUF]
    gv_sems = rest[2 * NBUF:3 * NBUF]
    st_sems = rest[3 * NBUF:4 * NBUF]

    wid = lax.axis_index("s") * NC + lax.axis_index("c")
    base = wid * N_PER_W

    pltpu.sync_copy(idx_hbm.at[pl.ds(base, N_PER_W)], idxm_v)
    pltpu.sync_copy(lo_hbm, lo_v)
    lo = lo_v[...]

    def split(j, carry):
        iv = idxm_v[pl.ds(j * LANES, LANES)]
        m = (iv >= lo) & (iv < lo + B)
        idxm_v[pl.ds(j * LANES, LANES)] = jnp.where(m, IGNORED, iv)
        idxv_v[pl.ds(j * LANES, LANES)] = jnp.where(m, iv - lo, IGNORED)
        return carry

    lax.fori_loop(0, N_PER_W // LANES, split, 0)

    def start_gathers(c):
        bi = c % NBUF
        sl = pl.ds(c * C, C)
        gm = pltpu.async_copy(
            mem_hbm.at[plsc.Indices(idxm_v.at[sl], ignored_value=IGNORED)],
            bufs[bi], gm_sems[bi])
        gv = pltpu.async_copy(
            val_hbm.at[plsc.Indices(idxv_v.at[sl], ignored_value=IGNORED)],
            bufs[bi], gv_sems[bi])
        return gm, gv

    def store_chunk(c):
        bi = c % NBUF
        return pltpu.async_copy(
            bufs[bi], out_hbm.at[pl.ds(base + c * C, C)], st_sems[bi])

    gm = [None] * NCHUNK
    gv = [None] * NCHUNK
    st = [None] * NCHUNK
    for c in range(NCHUNK):
        if c >= NBUF:
            st[c - NBUF].wait()
        gm[c], gv[c] = start_gathers(c)
        if c >= 1:
            gm[c - 1].wait()
            gv[c - 1].wait()
            st[c - 1] = store_chunk(c - 1)
    gm[NCHUNK - 1].wait()
    gv[NCHUNK - 1].wait()
    st[NCHUNK - 1] = store_chunk(NCHUNK - 1)
    for c in range(max(0, NCHUNK - NBUF), NCHUNK):
        st[c].wait()


@functools.partial(jax.jit, static_argnames=())
def kernel(mem, val, step, batch_idx):
    mem_flat = mem.reshape(TB, D)
    step = jnp.asarray(step, dtype=jnp.int32)
    lo_arr = jnp.full((LANES,), step * B, dtype=jnp.int32)

    run = pl.kernel(
        _body,
        out_type=jax.ShapeDtypeStruct((MB_ROWS, D), jnp.float32),
        mesh=plsc.VectorSubcoreMesh(core_axis_name="c", subcore_axis_name="s"),
        scratch_types=[
            pltpu.VMEM((N_PER_W,), jnp.int32),
            pltpu.VMEM((N_PER_W,), jnp.int32),
            pltpu.VMEM((LANES,), jnp.int32),
        ] + [pltpu.VMEM((C, D), jnp.float32)] * NBUF
          + [pltpu.SemaphoreType.DMA] * (3 * NBUF),
    )
    return run(mem_flat, val, lo_arr, batch_idx.astype(jnp.int32))


# speedup vs baseline: 1.0212x; 1.0212x over previous
import functools

import jax
import jax.numpy as jnp
from jax import lax
from jax.experimental import pallas as pl
from jax.experimental.pallas import tpu as pltpu
from jax.experimental.pallas import tpu_sc as plsc

T = 32
B = 4096
D = 128
TB = T * B
MB_ROWS = TB // 4

NC = 2
NS = 16
NW = NC * NS
N_PER_W = MB_ROWS // NW
C = 256
NCHUNK = N_PER_W // C
NBUF = 3
LANES = 16
IGNORED = -1


def _body(mem_hbm, val_hbm, lo_hbm, idx_hbm, out_hbm,
          idxm_v, idxv_v, lo_v, *rest):
    bufs = rest[:NBUF]
    gm_sems = rest[NBUF:2 * NBUF]
    gv_sems = rest[2 * NBUF:3 * NBUF]
    st_sems = rest[3 * NBUF:4 * NBUF]

    wid = lax.axis_index("s") * NC + lax.axis_index("c")
    base = wid * N_PER_W

    pltpu.sync_copy(idx_hbm.at[pl.ds(base, N_PER_W)], idxm_v)
    pltpu.sync_copy(lo_hbm, lo_v)
    lo = lo_v[...]

    def split_chunk(c):
        def split(j, carry):
            iv = idxm_v[pl.ds(j * LANES, LANES)]
            m = (iv >= lo) & (iv < lo + B)
            idxm_v[pl.ds(j * LANES, LANES)] = jnp.where(m, IGNORED, iv)
            idxv_v[pl.ds(j * LANES, LANES)] = jnp.where(m, iv - lo, IGNORED)
            return carry

        lax.fori_loop(c * (C // LANES), (c + 1) * (C // LANES), split, 0)

    def start_gathers(c):
        bi = c % NBUF
        sl = pl.ds(c * C, C)
        gm = pltpu.async_copy(
            mem_hbm.at[plsc.Indices(idxm_v.at[sl], ignored_value=IGNORED)],
            bufs[bi], gm_sems[bi])
        gv = pltpu.async_copy(
            val_hbm.at[plsc.Indices(idxv_v.at[sl], ignored_value=IGNORED)],
            bufs[bi], gv_sems[bi])
        return gm, gv

    def store_chunk(c):
        bi = c % NBUF
        return pltpu.async_copy(
            bufs[bi], out_hbm.at[pl.ds(base + c * C, C)], st_sems[bi])

    gm = [None] * NCHUNK
    gv = [None] * NCHUNK
    st = [None] * NCHUNK
    for c in range(NCHUNK):
        if c >= NBUF:
            st[c - NBUF].wait()
        split_chunk(c)
        gm[c], gv[c] = start_gathers(c)
        if c >= 1:
            gm[c - 1].wait()
            gv[c - 1].wait()
            st[c - 1] = store_chunk(c - 1)
    gm[NCHUNK - 1].wait()
    gv[NCHUNK - 1].wait()
    st[NCHUNK - 1] = store_chunk(NCHUNK - 1)
    for c in range(max(0, NCHUNK - NBUF), NCHUNK):
        st[c].wait()


@functools.partial(jax.jit, static_argnames=())
def kernel(mem, val, step, batch_idx):
    mem_flat = mem.reshape(TB, D)
    step = jnp.asarray(step, dtype=jnp.int32)
    lo_arr = jnp.full((LANES,), step * B, dtype=jnp.int32)

    run = pl.kernel(
        _body,
        out_type=jax.ShapeDtypeStruct((MB_ROWS, D), jnp.float32),
        mesh=plsc.VectorSubcoreMesh(core_axis_name="c", subcore_axis_name="s"),
        scratch_types=[
            pltpu.VMEM((N_PER_W,), jnp.int32),
            pltpu.VMEM((N_PER_W,), jnp.int32),
            pltpu.VMEM((LANES,), jnp.int32),
        ] + [pltpu.VMEM((C, D), jnp.float32)] * NBUF
          + [pltpu.SemaphoreType.DMA] * (3 * NBUF),
    )
    return run(mem_flat, val, lo_arr, batch_idx.astype(jnp.int32))


# speedup vs baseline: 1.0588x; 1.0368x over previous
import functools

import jax
import jax.numpy as jnp
from jax import lax
from jax.experimental import pallas as pl
from jax.experimental.pallas import tpu as pltpu
from jax.experimental.pallas import tpu_sc as plsc

T = 32
B = 4096
D = 128
TB = T * B
MB_ROWS = TB // 4

NC = 2
NS = 16
NW = NC * NS
N_PER_W = MB_ROWS // NW
C = 256
NCHUNK = N_PER_W // C
NBUF = 3
LANES = 16
IGNORED = -1


def _body(mem_hbm, val_hbm, lo_hbm, idx_hbm, out_hbm,
          idxm_v, idxv_v, lo_v, *rest):
    bufs = rest[:NBUF]
    gm_sems = rest[NBUF:2 * NBUF]
    gv_sems = rest[2 * NBUF:3 * NBUF]
    st_sems = rest[3 * NBUF:4 * NBUF]

    wid = lax.axis_index("s") * NC + lax.axis_index("c")
    base = wid * N_PER_W

    pltpu.sync_copy(idx_hbm.at[pl.ds(base, N_PER_W)], idxm_v)
    pltpu.sync_copy(lo_hbm, lo_v)
    lo = lo_v[...]

    def split(j, carry):
        iv = idxm_v[pl.ds(j * LANES, LANES)]
        m = (iv >= lo) & (iv < lo + B)
        idxm_v[pl.ds(j * LANES, LANES)] = jnp.where(m, IGNORED, iv)
        idxv_v[pl.ds(j * LANES, LANES)] = jnp.where(m, iv - lo, IGNORED)
        return carry

    lax.fori_loop(0, N_PER_W // LANES, split, 0)

    def start_gathers(c):
        bi = c % NBUF
        sl = pl.ds(c * C, C)
        gm = pltpu.async_copy(
            mem_hbm.at[plsc.Indices(idxm_v.at[sl], ignored_value=IGNORED)],
            bufs[bi], gm_sems[bi])
        gv = pltpu.async_copy(
            val_hbm.at[plsc.Indices(idxv_v.at[sl], ignored_value=IGNORED)],
            bufs[bi], gv_sems[bi])
        return gm, gv

    def store_chunk(c):
        bi = c % NBUF
        return pltpu.async_copy(
            bufs[bi], out_hbm.at[pl.ds(base + c * C, C)], st_sems[bi])

    gm = [None] * NCHUNK
    gv = [None] * NCHUNK
    st = [None] * NCHUNK
    for c in range(min(NBUF, NCHUNK)):
        gm[c], gv[c] = start_gathers(c)
    for c in range(NCHUNK):
        gm[c].wait()
        gv[c].wait()
        st[c] = store_chunk(c)
        nxt = c + NBUF
        if nxt < NCHUNK:
            st[c].wait()
            gm[nxt], gv[nxt] = start_gathers(nxt)
    for c in range(max(0, NCHUNK - NBUF), NCHUNK):
        st[c].wait()


@functools.partial(jax.jit, static_argnames=())
def kernel(mem, val, step, batch_idx):
    mem_flat = mem.reshape(TB, D)
    step = jnp.asarray(step, dtype=jnp.int32)
    lo_arr = jnp.full((LANES,), step * B, dtype=jnp.int32)

    run = pl.kernel(
        _body,
        out_type=jax.ShapeDtypeStruct((MB_ROWS, D), jnp.float32),
        mesh=plsc.VectorSubcoreMesh(core_axis_name="c", subcore_axis_name="s"),
        scratch_types=[
            pltpu.VMEM((N_PER_W,), jnp.int32),
            pltpu.VMEM((N_PER_W,), jnp.int32),
            pltpu.VMEM((LANES,), jnp.int32),
        ] + [pltpu.VMEM((C, D), jnp.float32)] * NBUF
          + [pltpu.SemaphoreType.DMA] * (3 * NBUF),
    )
    return run(mem_flat, val, lo_arr, batch_idx.astype(jnp.int32))
